# baseline (speedup 1.0000x reference)
import jax
import jax.numpy as jnp
from jax import lax
from jax.experimental import pallas as pl
from jax.experimental.pallas import tpu as pltpu

B, T, DIN, HID, OUT, IMG = 8, 128, 112, 256, 64, 128
FEAT = DIN + 16
COMB = FEAT + HID
G4 = 4 * HID
BT = B * T
NCH = 8
CH = BT // NCH
K1 = 3 * 11 * 128
M1 = 16 * 30
K2 = 16 * 3 * 16
M2 = 32 * 14


def _conv1_body(x_ref, w_ref, b_ref, out_ref):
    s = pl.program_id(1)
    strip = x_ref[:, pl.ds(4 * s, 11), :, :].reshape(K1, CH)
    acc = lax.dot_general(w_ref[...], strip, (((1,), (0,)), ((), ())),
                          preferred_element_type=jnp.float32)
    acc = jnp.maximum(acc + b_ref[...], 0.0)
    out_ref[...] = acc.astype(jnp.bfloat16).reshape(16, 30, CH)[:, None, :, :]


def _pool_phase(z, hw, ph):
    c = z.shape[0]
    pad = 2 * ph - hw
    if pad:
        z = jnp.concatenate(
            [z, jnp.full((c, pad, hw, CH), -jnp.inf, z.dtype)], axis=1)
        z = jnp.concatenate(
            [z, jnp.full((c, 2 * ph, pad, CH), -jnp.inf, z.dtype)], axis=2)
    z = z.reshape(c, ph, 2, ph, 2, CH)
    z00 = z[:, :, 0, :, 0, :]
    z01 = z[:, :, 0, :, 1, :]
    z10 = z[:, :, 1, :, 0, :]
    z11 = z[:, :, 1, :, 1, :]
    n = (hw - 3) // 2 + 1
    terms = [
        z00[:, 0:n, 0:n], z00[:, 0:n, 1:n + 1],
        z00[:, 1:n + 1, 0:n], z00[:, 1:n + 1, 1:n + 1],
        z01[:, 0:n, 0:n], z01[:, 1:n + 1, 0:n],
        z10[:, 0:n, 0:n], z10[:, 0:n, 1:n + 1],
        z11[:, 0:n, 0:n],
    ]
    r = terms[0]
    for t_ in terms[1:]:
        r = jnp.maximum(r, t_)
    return r


def _cnn_tail_body(y1_ref, w2_ref, b2_ref, wl_ref, bl_ref, feat_ref, y1p_ref):
    p1 = _pool_phase(y1_ref[...], 30, 15)
    y1p_ref[...] = jnp.zeros((16, 16, 16, CH), jnp.bfloat16)
    y1p_ref[:, 1:15, 1:15, :] = p1
    outs = []
    for i in range(14):
        strip = y1p_ref[:, pl.ds(i, 3), :, :].reshape(K2, CH)
        acc = lax.dot_general(w2_ref[...], strip, (((1,), (0,)), ((), ())),
                              preferred_element_type=jnp.float32)
        acc = jnp.maximum(acc + b2_ref[...], 0.0)
        outs.append(acc.astype(jnp.bfloat16).reshape(32, 1, 14, CH))
    y2 = jnp.concatenate(outs, axis=1)
    p2 = _pool_phase(y2, 14, 7)
    flat = p2.reshape(32 * 36, CH)
    f = lax.dot_general(wl_ref[...], flat, (((1,), (0,)), ((), ())),
                        preferred_element_type=jnp.float32)
    feat_ref[...] = jnp.maximum(f + bl_ref[...], 0.0)


def _lstm_body(lens_ref, x_ref, wx_ref, wh_ref, b_ref, wy_ref, by_ref,
               y_ref, h_out_ref, xproj_ref):
    xproj_ref[...] = jnp.dot(x_ref[...], wx_ref[...],
                             preferred_element_type=jnp.float32)
    lens = lens_ref[...]
    tmax = jnp.max(lens)
    wh = wh_ref[...]
    bias = b_ref[...]

    def step(t, carry):
        h, c = carry
        g = (xproj_ref[pl.ds(t * B, B), :]
             + jnp.dot(h, wh, preferred_element_type=jnp.float32)
             + bias)
        f = jax.nn.sigmoid(g[:, :HID])
        i = jax.nn.sigmoid(g[:, HID:2 * HID])
        cbar = jnp.tanh(g[:, 2 * HID:3 * HID])
        o = jax.nn.sigmoid(g[:, 3 * HID:])
        cn = f * c + i * cbar
        hn = o * jnp.tanh(cn)
        m = lens > t
        return (jnp.where(m, hn, h), jnp.where(m, cn, c))

    h0 = jnp.zeros((B, HID), jnp.float32)
    c0 = jnp.zeros((B, HID), jnp.float32)
    h, c = lax.fori_loop(0, tmax, step, (h0, c0))
    y_ref[...] = jnp.dot(h, wy_ref[...],
                         preferred_element_type=jnp.float32) + by_ref[...]
    h_out_ref[...] = h


def _build_toeplitz1(w):
    col = jnp.arange(128)[None, :]
    j = jnp.arange(30)[:, None]
    dx = col - 4 * j
    valid = (dx >= 0) & (dx <= 10)
    dxc = jnp.clip(dx, 0, 10)
    g = w[:, :, :, dxc]
    g = jnp.where(valid[None, None, None], g, 0.0)
    g = g.transpose(0, 3, 1, 2, 4).reshape(480, K1)
    return g.astype(jnp.bfloat16)


def _build_toeplitz2(w):
    col = jnp.arange(16)[None, :]
    j = jnp.arange(14)[:, None]
    dx = col - j
    valid = (dx >= 0) & (dx <= 2)
    dxc = jnp.clip(dx, 0, 2)
    g = w[:, :, :, dxc]
    g = jnp.where(valid[None, None, None], g, 0.0)
    g = g.transpose(0, 3, 1, 2, 4).reshape(448, K2)
    return g.astype(jnp.bfloat16)


@jax.jit
def kernel(datas, img, dataLens, conv1_w, conv1_b, conv2_w, conv2_b,
           lin_w, lin_b, Wf, bf, Wi, bi, Wc, bc, Wo, bo, Wy, by):
    x = img.reshape(BT, 3, IMG, IMG).transpose(1, 2, 3, 0).astype(jnp.bfloat16)
    w1t = _build_toeplitz1(conv1_w)
    b1v = jnp.tile(conv1_b[:, None], (1, 30)).reshape(M1, 1)
    w2t = _build_toeplitz2(conv2_w)
    b2v = jnp.tile(conv2_b[:, None], (1, 14)).reshape(M2, 1)
    wlt = lin_w.T.astype(jnp.bfloat16)
    blv = lin_b[:, None]

    y1 = pl.pallas_call(
        _conv1_body,
        grid=(NCH, 30),
        in_specs=[
            pl.BlockSpec((3, IMG, IMG, CH), lambda n, s: (0, 0, 0, n)),
            pl.BlockSpec((M1, K1), lambda n, s: (0, 0)),
            pl.BlockSpec((M1, 1), lambda n, s: (0, 0)),
        ],
        out_specs=pl.BlockSpec((16, 1, 30, CH), lambda n, s: (0, s, 0, n)),
        out_shape=jax.ShapeDtypeStruct((16, 30, 30, BT), jnp.bfloat16),
    )(x, w1t, b1v)

    feat_t = pl.pallas_call(
        _cnn_tail_body,
        grid=(NCH,),
        in_specs=[
            pl.BlockSpec((16, 30, 30, CH), lambda n: (0, 0, 0, n)),
            pl.BlockSpec((M2, K2), lambda n: (0, 0)),
            pl.BlockSpec((M2, 1), lambda n: (0, 0)),
            pl.BlockSpec((16, 32 * 36), lambda n: (0, 0)),
            pl.BlockSpec((16, 1), lambda n: (0, 0)),
        ],
        out_specs=pl.BlockSpec((16, CH), lambda n: (0, n)),
        out_shape=jax.ShapeDtypeStruct((16, BT), jnp.float32),
        scratch_shapes=[pltpu.VMEM((16, 16, 16, CH), jnp.bfloat16)],
    )(y1, w2t, b2v, wlt, blv)

    feat = feat_t.T.reshape(B, T, 16)
    xc = jnp.concatenate([feat, datas], axis=2)
    x_tm = xc.transpose(1, 0, 2).reshape(T * B, FEAT)

    wx = jnp.concatenate([Wf[:FEAT], Wi[:FEAT], Wc[:FEAT], Wo[:FEAT]], axis=1)
    wh = jnp.concatenate([Wf[FEAT:], Wi[FEAT:], Wc[FEAT:], Wo[FEAT:]], axis=1)
    bias = jnp.concatenate([bf, bi, bc, bo]).reshape(1, G4)
    lens = dataLens.astype(jnp.int32).reshape(B, 1)

    y, h = pl.pallas_call(
        _lstm_body,
        out_shape=[
            jax.ShapeDtypeStruct((B, OUT), jnp.float32),
            jax.ShapeDtypeStruct((B, HID), jnp.float32),
        ],
        scratch_shapes=[pltpu.VMEM((T * B, G4), jnp.float32)],
    )(lens, x_tm, wx, wh, bias, Wy, by.reshape(1, OUT))
    return (y, h)


# speedup vs baseline: 1.0690x; 1.0690x over previous
import jax
import jax.numpy as jnp
from jax import lax
from jax.experimental import pallas as pl
from jax.experimental.pallas import tpu as pltpu

B, T, DIN, HID, OUT, IMG = 8, 128, 112, 256, 64, 128
FEAT = DIN + 16
COMB = FEAT + HID
G4 = 4 * HID
BT = B * T
NCH = 8
CH = BT // NCH
K1 = 3 * 11 * 128
M1 = 16 * 30
K2 = 16 * 3 * 16
M2 = 32 * 14


def _conv1_body(x_ref, w_ref, b_ref, out_ref):
    s = pl.program_id(1)
    strip = x_ref[:, pl.ds(4 * s, 11), :, :].reshape(K1, CH)
    acc = lax.dot_general(w_ref[...], strip, (((1,), (0,)), ((), ())),
                          preferred_element_type=jnp.float32)
    acc = jnp.maximum(acc + b_ref[...], 0.0)
    out_ref[...] = acc.astype(jnp.bfloat16).reshape(16, 2, 15, CH)[:, None, None]


def _pool9(z00, z01, z10, z11, n):
    terms = [
        z00[:, 0:n, 0:n], z00[:, 0:n, 1:n + 1],
        z00[:, 1:n + 1, 0:n], z00[:, 1:n + 1, 1:n + 1],
        z01[:, 0:n, 0:n], z01[:, 1:n + 1, 0:n],
        z10[:, 0:n, 0:n], z10[:, 0:n, 1:n + 1],
        z11[:, 0:n, 0:n],
    ]
    r = terms[0]
    for t_ in terms[1:]:
        r = jnp.maximum(r, t_)
    return r


def _cnn_tail_body(y1_ref, w2_ref, b2_ref, wl_ref, bl_ref, feat_ref,
                   y1p_ref, y2_ref):
    p1 = _pool9(y1_ref[:, 0, :, 0], y1_ref[:, 0, :, 1],
                y1_ref[:, 1, :, 0], y1_ref[:, 1, :, 1], 14)
    y1p_ref[...] = jnp.zeros((16, 16, 16, CH), jnp.bfloat16)
    y1p_ref[:, 1:15, 1:15, :] = p1
    for i in range(14):
        strip = y1p_ref[:, i:i + 3, :, :].reshape(K2, CH)
        acc = lax.dot_general(w2_ref[...], strip, (((1,), (0,)), ((), ())),
                              preferred_element_type=jnp.float32)
        acc = jnp.maximum(acc + b2_ref[...], 0.0)
        y2_ref[:, i % 2, i // 2] = acc.astype(jnp.bfloat16).reshape(32, 2, 7, CH)
    p2 = _pool9(y2_ref[:, 0, :, 0], y2_ref[:, 0, :, 1],
                y2_ref[:, 1, :, 0], y2_ref[:, 1, :, 1], 6)
    flat = p2.reshape(32 * 36, CH)
    f = lax.dot_general(wl_ref[...], flat, (((1,), (0,)), ((), ())),
                        preferred_element_type=jnp.float32)
    feat_ref[...] = jnp.maximum(f + bl_ref[...], 0.0)


def _lstm_body(lens_ref, x_ref, wx_ref, wh_ref, b_ref, wy_ref, by_ref,
               y_ref, h_out_ref, xproj_ref):
    xproj_ref[...] = jnp.dot(x_ref[...], wx_ref[...],
                             preferred_element_type=jnp.float32)
    lens = lens_ref[...]
    tmax = jnp.max(lens)
    wh = wh_ref[...]
    bias = b_ref[...]

    def step(t, carry):
        h, c = carry
        g = (xproj_ref[pl.ds(t * B, B), :]
             + jnp.dot(h, wh, preferred_element_type=jnp.float32)
             + bias)
        f = jax.nn.sigmoid(g[:, :HID])
        i = jax.nn.sigmoid(g[:, HID:2 * HID])
        cbar = jnp.tanh(g[:, 2 * HID:3 * HID])
        o = jax.nn.sigmoid(g[:, 3 * HID:])
        cn = f * c + i * cbar
        hn = o * jnp.tanh(cn)
        m = lens > t
        return (jnp.where(m, hn, h), jnp.where(m, cn, c))

    h0 = jnp.zeros((B, HID), jnp.float32)
    c0 = jnp.zeros((B, HID), jnp.float32)
    h, c = lax.fori_loop(0, tmax, step, (h0, c0))
    y_ref[...] = jnp.dot(h, wy_ref[...],
                         preferred_element_type=jnp.float32) + by_ref[...]
    h_out_ref[...] = h


def _phase_perm(n):
    return jnp.concatenate([jnp.arange(0, n, 2), jnp.arange(1, n, 2)])


def _build_toeplitz1(w):
    col = jnp.arange(128)[None, :]
    j = jnp.arange(30)[:, None]
    dx = col - 4 * j
    valid = (dx >= 0) & (dx <= 10)
    dxc = jnp.clip(dx, 0, 10)
    g = w[:, :, :, dxc]
    g = jnp.where(valid[None, None, None], g, 0.0)
    g = g.transpose(0, 3, 1, 2, 4)
    g = g[:, _phase_perm(30)].reshape(480, K1)
    return g.astype(jnp.bfloat16)


def _build_toeplitz2(w):
    col = jnp.arange(16)[None, :]
    j = jnp.arange(14)[:, None]
    dx = col - j
    valid = (dx >= 0) & (dx <= 2)
    dxc = jnp.clip(dx, 0, 2)
    g = w[:, :, :, dxc]
    g = jnp.where(valid[None, None, None], g, 0.0)
    g = g.transpose(0, 3, 1, 2, 4)
    g = g[:, _phase_perm(14)].reshape(448, K2)
    return g.astype(jnp.bfloat16)


@jax.jit
def kernel(datas, img, dataLens, conv1_w, conv1_b, conv2_w, conv2_b,
           lin_w, lin_b, Wf, bf, Wi, bi, Wc, bc, Wo, bo, Wy, by):
    x = img.reshape(BT, 3, IMG, IMG).transpose(1, 2, 3, 0).astype(jnp.bfloat16)
    w1t = _build_toeplitz1(conv1_w)
    b1v = jnp.tile(conv1_b[:, None], (1, 30)).reshape(M1, 1)
    w2t = _build_toeplitz2(conv2_w)
    b2v = jnp.tile(conv2_b[:, None], (1, 14)).reshape(M2, 1)
    wlt = lin_w.T.astype(jnp.bfloat16)
    blv = lin_b[:, None]

    y1 = pl.pallas_call(
        _conv1_body,
        grid=(NCH, 30),
        in_specs=[
            pl.BlockSpec((3, IMG, IMG, CH), lambda n, s: (0, 0, 0, n)),
            pl.BlockSpec((M1, K1), lambda n, s: (0, 0)),
            pl.BlockSpec((M1, 1), lambda n, s: (0, 0)),
        ],
        out_specs=pl.BlockSpec((16, 1, 1, 2, 15, CH),
                               lambda n, s: (0, s % 2, s // 2, 0, 0, n)),
        out_shape=jax.ShapeDtypeStruct((16, 2, 15, 2, 15, BT), jnp.bfloat16),
    )(x, w1t, b1v)

    feat_t = pl.pallas_call(
        _cnn_tail_body,
        grid=(NCH,),
        in_specs=[
            pl.BlockSpec((16, 2, 15, 2, 15, CH), lambda n: (0, 0, 0, 0, 0, n)),
            pl.BlockSpec((M2, K2), lambda n: (0, 0)),
            pl.BlockSpec((M2, 1), lambda n: (0, 0)),
            pl.BlockSpec((16, 32 * 36), lambda n: (0, 0)),
            pl.BlockSpec((16, 1), lambda n: (0, 0)),
        ],
        out_specs=pl.BlockSpec((16, CH), lambda n: (0, n)),
        out_shape=jax.ShapeDtypeStruct((16, BT), jnp.float32),
        scratch_shapes=[pltpu.VMEM((16, 16, 16, CH), jnp.bfloat16),
                        pltpu.VMEM((32, 2, 7, 2, 7, CH), jnp.bfloat16)],
    )(y1, w2t, b2v, wlt, blv)

    feat = feat_t.T.reshape(B, T, 16)
    xc = jnp.concatenate([feat, datas], axis=2)
    x_tm = xc.transpose(1, 0, 2).reshape(T * B, FEAT)

    wx = jnp.concatenate([Wf[:FEAT], Wi[:FEAT], Wc[:FEAT], Wo[:FEAT]], axis=1)
    wh = jnp.concatenate([Wf[FEAT:], Wi[FEAT:], Wc[FEAT:], Wo[FEAT:]], axis=1)
    bias = jnp.concatenate([bf, bi, bc, bo]).reshape(1, G4)
    lens = dataLens.astype(jnp.int32).reshape(B, 1)

    y, h = pl.pallas_call(
        _lstm_body,
        out_shape=[
            jax.ShapeDtypeStruct((B, OUT), jnp.float32),
            jax.ShapeDtypeStruct((B, HID), jnp.float32),
        ],
        scratch_shapes=[pltpu.VMEM((T * B, G4), jnp.float32)],
    )(lens, x_tm, wx, wh, bias, Wy, by.reshape(1, OUT))
    return (y, h)


# speedup vs baseline: 1.3243x; 1.2388x over previous
import jax
import jax.numpy as jnp
from jax import lax
from jax.experimental import pallas as pl
from jax.experimental.pallas import tpu as pltpu

B, T, DIN, HID, OUT, IMG = 8, 128, 112, 256, 64, 128
FEAT = DIN + 16
COMB = FEAT + HID
G4 = 4 * HID
BT = B * T
NCH = 8
CH = BT // NCH
K1G = 3 * 11 * 64
K2 = 16 * 3 * 16
M2 = 32 * 14


def _conv1_body(x_ref, w_ref, b_ref, out_ref):
    s = pl.program_id(1)
    parts = []
    for g in range(4):
        strip = x_ref[:, pl.ds(4 * s, 11),
                      32 * g:32 * g + 64, :].reshape(K1G, CH)
        acc = lax.dot_general(w_ref[...], strip, (((1,), (0,)), ((), ())),
                              preferred_element_type=jnp.float32)
        acc = jnp.maximum(acc + b_ref[...], 0.0)
        parts.append(acc.astype(jnp.bfloat16).reshape(16, 2, 4, CH))
    out_ref[...] = jnp.concatenate(parts, axis=2)[:, None, None]


def _pool9(z00, z01, z10, z11, n):
    terms = [
        z00[:, 0:n, 0:n], z00[:, 0:n, 1:n + 1],
        z00[:, 1:n + 1, 0:n], z00[:, 1:n + 1, 1:n + 1],
        z01[:, 0:n, 0:n], z01[:, 1:n + 1, 0:n],
        z10[:, 0:n, 0:n], z10[:, 0:n, 1:n + 1],
        z11[:, 0:n, 0:n],
    ]
    r = terms[0]
    for t_ in terms[1:]:
        r = jnp.maximum(r, t_)
    return r


def _cnn_tail_body(y1_ref, w2_ref, b2_ref, wl_ref, bl_ref, feat_ref,
                   y1p_ref, y2_ref):
    p1 = _pool9(y1_ref[:, 0, :, 0], y1_ref[:, 0, :, 1],
                y1_ref[:, 1, :, 0], y1_ref[:, 1, :, 1], 14)
    y1p_ref[...] = jnp.zeros((16, 16, 16, CH), jnp.bfloat16)
    y1p_ref[:, 1:15, 1:15, :] = p1
    for i in range(14):
        strip = y1p_ref[:, i:i + 3, :, :].reshape(K2, CH)
        acc = lax.dot_general(w2_ref[...], strip, (((1,), (0,)), ((), ())),
                              preferred_element_type=jnp.float32)
        acc = jnp.maximum(acc + b2_ref[...], 0.0)
        y2_ref[:, i % 2, i // 2] = acc.astype(jnp.bfloat16).reshape(32, 2, 7, CH)
    p2 = _pool9(y2_ref[:, 0, :, 0], y2_ref[:, 0, :, 1],
                y2_ref[:, 1, :, 0], y2_ref[:, 1, :, 1], 6)
    flat = p2.reshape(32 * 36, CH)
    f = lax.dot_general(wl_ref[...], flat, (((1,), (0,)), ((), ())),
                        preferred_element_type=jnp.float32)
    feat_ref[...] = jnp.maximum(f + bl_ref[...], 0.0)


def _lstm_body(lens_ref, x_ref, wx_ref, wh_ref, b_ref, wy_ref, by_ref,
               y_ref, h_out_ref, xproj_ref):
    xproj_ref[...] = jnp.dot(x_ref[...], wx_ref[...],
                             preferred_element_type=jnp.float32)
    lens = lens_ref[...]
    tmax = jnp.max(lens)
    wh = wh_ref[...]
    bias = b_ref[...]

    def step(t, carry):
        h, c = carry
        g = (xproj_ref[pl.ds(t * B, B), :]
             + jnp.dot(h, wh, preferred_element_type=jnp.float32)
             + bias)
        f = jax.nn.sigmoid(g[:, :HID])
        i = jax.nn.sigmoid(g[:, HID:2 * HID])
        cbar = jnp.tanh(g[:, 2 * HID:3 * HID])
        o = jax.nn.sigmoid(g[:, 3 * HID:])
        cn = f * c + i * cbar
        hn = o * jnp.tanh(cn)
        m = lens > t
        return (jnp.where(m, hn, h), jnp.where(m, cn, c))

    h0 = jnp.zeros((B, HID), jnp.float32)
    c0 = jnp.zeros((B, HID), jnp.float32)
    h, c = lax.fori_loop(0, tmax, step, (h0, c0))
    y_ref[...] = jnp.dot(h, wy_ref[...],
                         preferred_element_type=jnp.float32) + by_ref[...]
    h_out_ref[...] = h


def _phase_perm(n):
    return jnp.concatenate([jnp.arange(0, n, 2), jnp.arange(1, n, 2)])


def _build_toeplitz1(w):
    col = jnp.arange(64)[None, :]
    l = jnp.arange(8)[:, None]
    dx = col - 4 * l
    valid = (dx >= 0) & (dx <= 10)
    dxc = jnp.clip(dx, 0, 10)
    g = w[:, :, :, dxc]
    g = jnp.where(valid[None, None, None], g, 0.0)
    g = g.transpose(0, 3, 1, 2, 4)
    g = g[:, _phase_perm(8)].reshape(128, K1G)
    return g.astype(jnp.bfloat16)


def _build_toeplitz2(w):
    col = jnp.arange(16)[None, :]
    j = jnp.arange(14)[:, None]
    dx = col - j
    valid = (dx >= 0) & (dx <= 2)
    dxc = jnp.clip(dx, 0, 2)
    g = w[:, :, :, dxc]
    g = jnp.where(valid[None, None, None], g, 0.0)
    g = g.transpose(0, 3, 1, 2, 4)
    g = g[:, _phase_perm(14)].reshape(448, K2)
    return g.astype(jnp.bfloat16)


@jax.jit
def kernel(datas, img, dataLens, conv1_w, conv1_b, conv2_w, conv2_b,
           lin_w, lin_b, Wf, bf, Wi, bi, Wc, bc, Wo, bo, Wy, by):
    x = img.reshape(BT, 3, IMG, IMG).transpose(1, 2, 3, 0).astype(jnp.bfloat16)
    x = jnp.pad(x, ((0, 0), (0, 0), (0, 32), (0, 0)))
    w1t = _build_toeplitz1(conv1_w)
    b1v = jnp.tile(conv1_b[:, None], (1, 8)).reshape(128, 1)
    w2t = _build_toeplitz2(conv2_w)
    b2v = jnp.tile(conv2_b[:, None], (1, 14)).reshape(M2, 1)
    wlt = lin_w.T.astype(jnp.bfloat16)
    blv = lin_b[:, None]

    y1 = pl.pallas_call(
        _conv1_body,
        grid=(NCH, 30),
        in_specs=[
            pl.BlockSpec((3, IMG, 160, CH), lambda n, s: (0, 0, 0, n)),
            pl.BlockSpec((128, K1G), lambda n, s: (0, 0)),
            pl.BlockSpec((128, 1), lambda n, s: (0, 0)),
        ],
        out_specs=pl.BlockSpec((16, 1, 1, 2, 16, CH),
                               lambda n, s: (0, s % 2, s // 2, 0, 0, n)),
        out_shape=jax.ShapeDtypeStruct((16, 2, 15, 2, 16, BT), jnp.bfloat16),
    )(x, w1t, b1v)

    feat_t = pl.pallas_call(
        _cnn_tail_body,
        grid=(NCH,),
        in_specs=[
            pl.BlockSpec((16, 2, 15, 2, 16, CH), lambda n: (0, 0, 0, 0, 0, n)),
            pl.BlockSpec((M2, K2), lambda n: (0, 0)),
            pl.BlockSpec((M2, 1), lambda n: (0, 0)),
            pl.BlockSpec((16, 32 * 36), lambda n: (0, 0)),
            pl.BlockSpec((16, 1), lambda n: (0, 0)),
        ],
        out_specs=pl.BlockSpec((16, CH), lambda n: (0, n)),
        out_shape=jax.ShapeDtypeStruct((16, BT), jnp.float32),
        scratch_shapes=[pltpu.VMEM((16, 16, 16, CH), jnp.bfloat16),
                        pltpu.VMEM((32, 2, 7, 2, 7, CH), jnp.bfloat16)],
    )(y1, w2t, b2v, wlt, blv)

    feat = feat_t.T.reshape(B, T, 16)
    xc = jnp.concatenate([feat, datas], axis=2)
    x_tm = xc.transpose(1, 0, 2).reshape(T * B, FEAT)

    wx = jnp.concatenate([Wf[:FEAT], Wi[:FEAT], Wc[:FEAT], Wo[:FEAT]], axis=1)
    wh = jnp.concatenate([Wf[FEAT:], Wi[FEAT:], Wc[FEAT:], Wo[FEAT:]], axis=1)
    bias = jnp.concatenate([bf, bi, bc, bo]).reshape(1, G4)
    lens = dataLens.astype(jnp.int32).reshape(B, 1)

    y, h = pl.pallas_call(
        _lstm_body,
        out_shape=[
            jax.ShapeDtypeStruct((B, OUT), jnp.float32),
            jax.ShapeDtypeStruct((B, HID), jnp.float32),
        ],
        scratch_shapes=[pltpu.VMEM((T * B, G4), jnp.float32)],
    )(lens, x_tm, wx, wh, bias, Wy, by.reshape(1, OUT))
    return (y, h)


# speedup vs baseline: 1.4778x; 1.1160x over previous
import jax
import jax.numpy as jnp
from jax import lax
from jax.experimental import pallas as pl
from jax.experimental.pallas import tpu as pltpu

B, T, DIN, HID, OUT, IMG = 8, 128, 112, 256, 64, 128
FEAT = DIN + 16
COMB = FEAT + HID
G4 = 4 * HID
BT = B * T
NCH = 8
CH = BT // NCH
K1G = 3 * 11 * 40
K2 = 16 * 3 * 16
M2 = 32 * 14


def _conv1_body(x_ref, w_ref, b_ref, out_ref):
    s = pl.program_id(1)
    parts = []
    for g in range(4):
        strip = x_ref[:, pl.ds(4 * s, 11),
                      32 * g:32 * g + 40, :].reshape(K1G, CH)
        acc = lax.dot_general(w_ref[...], strip, (((1,), (0,)), ((), ())),
                              preferred_element_type=jnp.float32)
        acc = jnp.maximum(acc + b_ref[...], 0.0)
        parts.append(acc.astype(jnp.bfloat16).reshape(16, 2, 4, CH))
    out_ref[...] = jnp.concatenate(parts, axis=2)[:, None, None]


def _pool9(z00, z01, z10, z11, n):
    terms = [
        z00[:, 0:n, 0:n], z00[:, 0:n, 1:n + 1],
        z00[:, 1:n + 1, 0:n], z00[:, 1:n + 1, 1:n + 1],
        z01[:, 0:n, 0:n], z01[:, 1:n + 1, 0:n],
        z10[:, 0:n, 0:n], z10[:, 0:n, 1:n + 1],
        z11[:, 0:n, 0:n],
    ]
    r = terms[0]
    for t_ in terms[1:]:
        r = jnp.maximum(r, t_)
    return r


def _cnn_tail_body(y1_ref, w2_ref, b2_ref, wl_ref, bl_ref, feat_ref,
                   y1p_ref, y2_ref):
    p1 = _pool9(y1_ref[:, 0, :, 0], y1_ref[:, 0, :, 1],
                y1_ref[:, 1, :, 0], y1_ref[:, 1, :, 1], 14)
    y1p_ref[...] = jnp.zeros((16, 16, 16, CH), jnp.bfloat16)
    y1p_ref[:, 1:15, 1:15, :] = p1
    for i in range(14):
        strip = y1p_ref[:, i:i + 3, :, :].reshape(K2, CH)
        acc = lax.dot_general(w2_ref[...], strip, (((1,), (0,)), ((), ())),
                              preferred_element_type=jnp.float32)
        acc = jnp.maximum(acc + b2_ref[...], 0.0)
        y2_ref[:, i % 2, i // 2] = acc.astype(jnp.bfloat16).reshape(32, 2, 7, CH)
    p2 = _pool9(y2_ref[:, 0, :, 0], y2_ref[:, 0, :, 1],
                y2_ref[:, 1, :, 0], y2_ref[:, 1, :, 1], 6)
    flat = p2.reshape(32 * 36, CH)
    f = lax.dot_general(wl_ref[...], flat, (((1,), (0,)), ((), ())),
                        preferred_element_type=jnp.float32)
    feat_ref[...] = jnp.maximum(f + bl_ref[...], 0.0)


def _lstm_body(lens_ref, x_ref, wx_ref, wh_ref, b_ref, wy_ref, by_ref,
               y_ref, h_out_ref, xproj_ref):
    xproj_ref[...] = jnp.dot(x_ref[...], wx_ref[...],
                             preferred_element_type=jnp.float32)
    lens = lens_ref[...]
    tmax = jnp.max(lens)
    wh = wh_ref[...]
    bias = b_ref[...]

    def step(t, carry):
        h, c = carry
        g = (xproj_ref[pl.ds(t * B, B), :]
             + jnp.dot(h, wh, preferred_element_type=jnp.float32)
             + bias)
        f = jax.nn.sigmoid(g[:, :HID])
        i = jax.nn.sigmoid(g[:, HID:2 * HID])
        cbar = jnp.tanh(g[:, 2 * HID:3 * HID])
        o = jax.nn.sigmoid(g[:, 3 * HID:])
        cn = f * c + i * cbar
        hn = o * jnp.tanh(cn)
        m = lens > t
        return (jnp.where(m, hn, h), jnp.where(m, cn, c))

    h0 = jnp.zeros((B, HID), jnp.float32)
    c0 = jnp.zeros((B, HID), jnp.float32)
    h, c = lax.fori_loop(0, tmax, step, (h0, c0))
    y_ref[...] = jnp.dot(h, wy_ref[...],
                         preferred_element_type=jnp.float32) + by_ref[...]
    h_out_ref[...] = h


def _phase_perm(n):
    return jnp.concatenate([jnp.arange(0, n, 2), jnp.arange(1, n, 2)])


def _build_toeplitz1(w):
    col = jnp.arange(40)[None, :]
    l = jnp.arange(8)[:, None]
    dx = col - 4 * l
    valid = (dx >= 0) & (dx <= 10)
    dxc = jnp.clip(dx, 0, 10)
    g = w[:, :, :, dxc]
    g = jnp.where(valid[None, None, None], g, 0.0)
    g = g.transpose(0, 3, 1, 2, 4)
    g = g[:, _phase_perm(8)].reshape(128, K1G)
    return g.astype(jnp.bfloat16)


def _build_toeplitz2(w):
    col = jnp.arange(16)[None, :]
    j = jnp.arange(14)[:, None]
    dx = col - j
    valid = (dx >= 0) & (dx <= 2)
    dxc = jnp.clip(dx, 0, 2)
    g = w[:, :, :, dxc]
    g = jnp.where(valid[None, None, None], g, 0.0)
    g = g.transpose(0, 3, 1, 2, 4)
    g = g[:, _phase_perm(14)].reshape(448, K2)
    return g.astype(jnp.bfloat16)


@jax.jit
def kernel(datas, img, dataLens, conv1_w, conv1_b, conv2_w, conv2_b,
           lin_w, lin_b, Wf, bf, Wi, bi, Wc, bc, Wo, bo, Wy, by):
    x = img.reshape(BT, 3, IMG, IMG).transpose(1, 2, 3, 0).astype(jnp.bfloat16)
    x = jnp.pad(x, ((0, 0), (0, 0), (0, 8), (0, 0)))
    w1t = _build_toeplitz1(conv1_w)
    b1v = jnp.tile(conv1_b[:, None], (1, 8)).reshape(128, 1)
    w2t = _build_toeplitz2(conv2_w)
    b2v = jnp.tile(conv2_b[:, None], (1, 14)).reshape(M2, 1)
    wlt = lin_w.T.astype(jnp.bfloat16)
    blv = lin_b[:, None]

    y1 = pl.pallas_call(
        _conv1_body,
        grid=(NCH, 30),
        in_specs=[
            pl.BlockSpec((3, IMG, 136, CH), lambda n, s: (0, 0, 0, n)),
            pl.BlockSpec((128, K1G), lambda n, s: (0, 0)),
            pl.BlockSpec((128, 1), lambda n, s: (0, 0)),
        ],
        out_specs=pl.BlockSpec((16, 1, 1, 2, 16, CH),
                               lambda n, s: (0, s % 2, s // 2, 0, 0, n)),
        out_shape=jax.ShapeDtypeStruct((16, 2, 15, 2, 16, BT), jnp.bfloat16),
    )(x, w1t, b1v)

    feat_t = pl.pallas_call(
        _cnn_tail_body,
        grid=(NCH,),
        in_specs=[
            pl.BlockSpec((16, 2, 15, 2, 16, CH), lambda n: (0, 0, 0, 0, 0, n)),
            pl.BlockSpec((M2, K2), lambda n: (0, 0)),
            pl.BlockSpec((M2, 1), lambda n: (0, 0)),
            pl.BlockSpec((16, 32 * 36), lambda n: (0, 0)),
            pl.BlockSpec((16, 1), lambda n: (0, 0)),
        ],
        out_specs=pl.BlockSpec((16, CH), lambda n: (0, n)),
        out_shape=jax.ShapeDtypeStruct((16, BT), jnp.float32),
        scratch_shapes=[pltpu.VMEM((16, 16, 16, CH), jnp.bfloat16),
                        pltpu.VMEM((32, 2, 7, 2, 7, CH), jnp.bfloat16)],
    )(y1, w2t, b2v, wlt, blv)

    feat = feat_t.T.reshape(B, T, 16)
    xc = jnp.concatenate([feat, datas], axis=2)
    x_tm = xc.transpose(1, 0, 2).reshape(T * B, FEAT)

    wx = jnp.concatenate([Wf[:FEAT], Wi[:FEAT], Wc[:FEAT], Wo[:FEAT]], axis=1)
    wh = jnp.concatenate([Wf[FEAT:], Wi[FEAT:], Wc[FEAT:], Wo[FEAT:]], axis=1)
    bias = jnp.concatenate([bf, bi, bc, bo]).reshape(1, G4)
    lens = dataLens.astype(jnp.int32).reshape(B, 1)

    y, h = pl.pallas_call(
        _lstm_body,
        out_shape=[
            jax.ShapeDtypeStruct((B, OUT), jnp.float32),
            jax.ShapeDtypeStruct((B, HID), jnp.float32),
        ],
        scratch_shapes=[pltpu.VMEM((T * B, G4), jnp.float32)],
    )(lens, x_tm, wx, wh, bias, Wy, by.reshape(1, OUT))
    return (y, h)


# speedup vs baseline: 1.6489x; 1.1157x over previous
import jax
import jax.numpy as jnp
from jax import lax
from jax.experimental import pallas as pl
from jax.experimental.pallas import tpu as pltpu

B, T, DIN, HID, OUT, IMG = 8, 128, 112, 256, 64, 128
FEAT = DIN + 16
COMB = FEAT + HID
G4 = 4 * HID
BT = B * T
NCH = 8
CH = BT // NCH
K1G = 3 * 11 * 40
K2 = 16 * 3 * 16
M2 = 32 * 14


def _conv1_body(x_ref, w_ref, wb_ref, b_ref, out_ref):
    s = pl.program_id(1)
    parts = []
    for g in range(4):
        width = 40 if g < 3 else 32
        wg = w_ref if g < 3 else wb_ref
        strip = x_ref[:, pl.ds(4 * s, 11),
                      32 * g:32 * g + width, :].reshape(3 * 11 * width, CH)
        acc = lax.dot_general(wg[...], strip, (((1,), (0,)), ((), ())),
                              preferred_element_type=jnp.float32)
        acc = jnp.maximum(acc + b_ref[...], 0.0)
        parts.append(acc.astype(jnp.bfloat16).reshape(16, 2, 4, CH))
    out_ref[...] = jnp.concatenate(parts, axis=2)[:, None, None]


def _pool9(z00, z01, z10, z11, n):
    terms = [
        z00[:, 0:n, 0:n], z00[:, 0:n, 1:n + 1],
        z00[:, 1:n + 1, 0:n], z00[:, 1:n + 1, 1:n + 1],
        z01[:, 0:n, 0:n], z01[:, 1:n + 1, 0:n],
        z10[:, 0:n, 0:n], z10[:, 0:n, 1:n + 1],
        z11[:, 0:n, 0:n],
    ]
    r = terms[0]
    for t_ in terms[1:]:
        r = jnp.maximum(r, t_)
    return r


def _cnn_tail_body(y1_ref, w2_ref, b2_ref, wl_ref, bl_ref, feat_ref,
                   y1p_ref, y2_ref):
    p1 = _pool9(y1_ref[:, 0, :, 0], y1_ref[:, 0, :, 1],
                y1_ref[:, 1, :, 0], y1_ref[:, 1, :, 1], 14)
    y1p_ref[...] = jnp.zeros((16, 16, 16, CH), jnp.bfloat16)
    y1p_ref[:, 1:15, 1:15, :] = p1
    for i in range(14):
        strip = y1p_ref[:, i:i + 3, :, :].reshape(K2, CH)
        acc = lax.dot_general(w2_ref[...], strip, (((1,), (0,)), ((), ())),
                              preferred_element_type=jnp.float32)
        acc = jnp.maximum(acc + b2_ref[...], 0.0)
        y2_ref[:, i % 2, i // 2] = acc.astype(jnp.bfloat16).reshape(32, 2, 7, CH)
    p2 = _pool9(y2_ref[:, 0, :, 0], y2_ref[:, 0, :, 1],
                y2_ref[:, 1, :, 0], y2_ref[:, 1, :, 1], 6)
    flat = p2.reshape(32 * 36, CH)
    f = lax.dot_general(wl_ref[...], flat, (((1,), (0,)), ((), ())),
                        preferred_element_type=jnp.float32)
    feat_ref[...] = jnp.maximum(f + bl_ref[...], 0.0)


def _lstm_body(lens_ref, x_ref, wx_ref, wh_ref, b_ref, wy_ref, by_ref,
               y_ref, h_out_ref, xproj_ref):
    xproj_ref[...] = jnp.dot(x_ref[...], wx_ref[...],
                             preferred_element_type=jnp.float32)
    lens = lens_ref[...]
    tmax = jnp.max(lens)
    wh = wh_ref[...]
    bias = b_ref[...]

    def step(t, carry):
        h, c = carry
        g = (xproj_ref[pl.ds(t * B, B), :]
             + jnp.dot(h, wh, preferred_element_type=jnp.float32)
             + bias)
        f = jax.nn.sigmoid(g[:, :HID])
        i = jax.nn.sigmoid(g[:, HID:2 * HID])
        cbar = jnp.tanh(g[:, 2 * HID:3 * HID])
        o = jax.nn.sigmoid(g[:, 3 * HID:])
        cn = f * c + i * cbar
        hn = o * jnp.tanh(cn)
        m = lens > t
        return (jnp.where(m, hn, h), jnp.where(m, cn, c))

    h0 = jnp.zeros((B, HID), jnp.float32)
    c0 = jnp.zeros((B, HID), jnp.float32)
    h, c = lax.fori_loop(0, tmax, step, (h0, c0))
    y_ref[...] = jnp.dot(h, wy_ref[...],
                         preferred_element_type=jnp.float32) + by_ref[...]
    h_out_ref[...] = h


def _phase_perm(n):
    return jnp.concatenate([jnp.arange(0, n, 2), jnp.arange(1, n, 2)])


def _build_toeplitz1(w, width=40):
    col = jnp.arange(width)[None, :]
    l = jnp.arange(8)[:, None]
    dx = col - 4 * l
    valid = (dx >= 0) & (dx <= 10)
    dxc = jnp.clip(dx, 0, 10)
    g = w[:, :, :, dxc]
    g = jnp.where(valid[None, None, None], g, 0.0)
    g = g.transpose(0, 3, 1, 2, 4)
    g = g[:, _phase_perm(8)].reshape(128, 3 * 11 * width)
    return g.astype(jnp.bfloat16)


def _build_toeplitz2(w):
    col = jnp.arange(16)[None, :]
    j = jnp.arange(14)[:, None]
    dx = col - j
    valid = (dx >= 0) & (dx <= 2)
    dxc = jnp.clip(dx, 0, 2)
    g = w[:, :, :, dxc]
    g = jnp.where(valid[None, None, None], g, 0.0)
    g = g.transpose(0, 3, 1, 2, 4)
    g = g[:, _phase_perm(14)].reshape(448, K2)
    return g.astype(jnp.bfloat16)


@jax.jit
def kernel(datas, img, dataLens, conv1_w, conv1_b, conv2_w, conv2_b,
           lin_w, lin_b, Wf, bf, Wi, bi, Wc, bc, Wo, bo, Wy, by):
    x = img.reshape(BT, 3, IMG, IMG).transpose(1, 2, 3, 0).astype(jnp.bfloat16)
    w1t = _build_toeplitz1(conv1_w, 40)
    w1tb = _build_toeplitz1(conv1_w, 32)
    b1v = jnp.tile(conv1_b[:, None], (1, 8)).reshape(128, 1)
    w2t = _build_toeplitz2(conv2_w)
    b2v = jnp.tile(conv2_b[:, None], (1, 14)).reshape(M2, 1)
    wlt = lin_w.T.astype(jnp.bfloat16)
    blv = lin_b[:, None]

    y1 = pl.pallas_call(
        _conv1_body,
        grid=(NCH, 30),
        in_specs=[
            pl.BlockSpec((3, IMG, IMG, CH), lambda n, s: (0, 0, 0, n)),
            pl.BlockSpec((128, K1G), lambda n, s: (0, 0)),
            pl.BlockSpec((128, 3 * 11 * 32), lambda n, s: (0, 0)),
            pl.BlockSpec((128, 1), lambda n, s: (0, 0)),
        ],
        out_specs=pl.BlockSpec((16, 1, 1, 2, 16, CH),
                               lambda n, s: (0, s % 2, s // 2, 0, 0, n)),
        out_shape=jax.ShapeDtypeStruct((16, 2, 15, 2, 16, BT), jnp.bfloat16),
    )(x, w1t, w1tb, b1v)

    feat_t = pl.pallas_call(
        _cnn_tail_body,
        grid=(NCH,),
        in_specs=[
            pl.BlockSpec((16, 2, 15, 2, 16, CH), lambda n: (0, 0, 0, 0, 0, n)),
            pl.BlockSpec((M2, K2), lambda n: (0, 0)),
            pl.BlockSpec((M2, 1), lambda n: (0, 0)),
            pl.BlockSpec((16, 32 * 36), lambda n: (0, 0)),
            pl.BlockSpec((16, 1), lambda n: (0, 0)),
        ],
        out_specs=pl.BlockSpec((16, CH), lambda n: (0, n)),
        out_shape=jax.ShapeDtypeStruct((16, BT), jnp.float32),
        scratch_shapes=[pltpu.VMEM((16, 16, 16, CH), jnp.bfloat16),
                        pltpu.VMEM((32, 2, 7, 2, 7, CH), jnp.bfloat16)],
    )(y1, w2t, b2v, wlt, blv)

    feat = feat_t.T.reshape(B, T, 16)
    xc = jnp.concatenate([feat, datas], axis=2)
    x_tm = xc.transpose(1, 0, 2).reshape(T * B, FEAT)

    wx = jnp.concatenate([Wf[:FEAT], Wi[:FEAT], Wc[:FEAT], Wo[:FEAT]], axis=1)
    wh = jnp.concatenate([Wf[FEAT:], Wi[FEAT:], Wc[FEAT:], Wo[FEAT:]], axis=1)
    bias = jnp.concatenate([bf, bi, bc, bo]).reshape(1, G4)
    lens = dataLens.astype(jnp.int32).reshape(B, 1)

    y, h = pl.pallas_call(
        _lstm_body,
        out_shape=[
            jax.ShapeDtypeStruct((B, OUT), jnp.float32),
            jax.ShapeDtypeStruct((B, HID), jnp.float32),
        ],
        scratch_shapes=[pltpu.VMEM((T * B, G4), jnp.float32)],
    )(lens, x_tm, wx, wh, bias, Wy, by.reshape(1, OUT))
    return (y, h)


# speedup vs baseline: 1.7362x; 1.0530x over previous
import jax
import jax.numpy as jnp
from jax import lax
from jax.experimental import pallas as pl
from jax.experimental.pallas import tpu as pltpu

B, T, DIN, HID, OUT, IMG = 8, 128, 112, 256, 64, 128
FEAT = DIN + 16
COMB = FEAT + HID
G4 = 4 * HID
BT = B * T
NCH = 8
CH = BT // NCH
NCHT = 4
CHT = BT // NCHT
K1G = 3 * 11 * 40
K2 = 16 * 3 * 16
M2 = 32 * 14


def _conv1_body(x_ref, w_ref, wb_ref, b_ref, out_ref):
    s = pl.program_id(1)
    parts = []
    for g in range(4):
        width = 40 if g < 3 else 32
        wg = w_ref if g < 3 else wb_ref
        strip = x_ref[:, pl.ds(4 * s, 11),
                      32 * g:32 * g + width, :].reshape(3 * 11 * width, CH)
        acc = lax.dot_general(wg[...], strip, (((1,), (0,)), ((), ())),
                              preferred_element_type=jnp.float32)
        acc = jnp.maximum(acc + b_ref[...], 0.0)
        parts.append(acc.astype(jnp.bfloat16).reshape(16, 2, 4, CH))
    out_ref[...] = jnp.concatenate(parts, axis=2)[:, None, None]


def _pool9(z00, z01, z10, z11, n):
    terms = [
        z00[:, 0:n, 0:n], z00[:, 0:n, 1:n + 1],
        z00[:, 1:n + 1, 0:n], z00[:, 1:n + 1, 1:n + 1],
        z01[:, 0:n, 0:n], z01[:, 1:n + 1, 0:n],
        z10[:, 0:n, 0:n], z10[:, 0:n, 1:n + 1],
        z11[:, 0:n, 0:n],
    ]
    r = terms[0]
    for t_ in terms[1:]:
        r = jnp.maximum(r, t_)
    return r


def _cnn_tail_body(y1_ref, w2_ref, b2_ref, wl_ref, bl_ref, feat_ref,
                   y1p_ref, y2_ref):
    p1 = _pool9(y1_ref[:, 0, :, 0], y1_ref[:, 0, :, 1],
                y1_ref[:, 1, :, 0], y1_ref[:, 1, :, 1], 14)
    y1p_ref[...] = jnp.zeros((16, 16, 16, CHT), jnp.bfloat16)
    y1p_ref[:, 1:15, 1:15, :] = p1
    for i in range(14):
        strip = y1p_ref[:, i:i + 3, :, :].reshape(K2, CHT)
        acc = lax.dot_general(w2_ref[...], strip, (((1,), (0,)), ((), ())),
                              preferred_element_type=jnp.float32)
        acc = jnp.maximum(acc + b2_ref[...], 0.0)
        y2_ref[:, i % 2, i // 2] = acc.astype(jnp.bfloat16).reshape(32, 2, 7, CHT)
    p2 = _pool9(y2_ref[:, 0, :, 0], y2_ref[:, 0, :, 1],
                y2_ref[:, 1, :, 0], y2_ref[:, 1, :, 1], 6)
    flat = p2.reshape(32 * 36, CHT)
    f = lax.dot_general(wl_ref[...], flat, (((1,), (0,)), ((), ())),
                        preferred_element_type=jnp.float32)
    feat_ref[...] = jnp.maximum(f + bl_ref[...], 0.0)


def _lstm_body(lens_ref, x_ref, wx_ref, wh_ref, b_ref, wy_ref, by_ref,
               y_ref, h_out_ref, xproj_ref):
    xproj_ref[...] = jnp.dot(x_ref[...], wx_ref[...],
                             preferred_element_type=jnp.float32)
    lens = lens_ref[...]
    tmax = jnp.max(lens)
    wh = wh_ref[...]
    bias = b_ref[...]

    def step(t, carry):
        h, c = carry
        g = (xproj_ref[pl.ds(t * B, B), :]
             + jnp.dot(h, wh, preferred_element_type=jnp.float32)
             + bias)
        f = jax.nn.sigmoid(g[:, :HID])
        i = jax.nn.sigmoid(g[:, HID:2 * HID])
        cbar = jnp.tanh(g[:, 2 * HID:3 * HID])
        o = jax.nn.sigmoid(g[:, 3 * HID:])
        cn = f * c + i * cbar
        hn = o * jnp.tanh(cn)
        m = lens > t
        return (jnp.where(m, hn, h), jnp.where(m, cn, c))

    h0 = jnp.zeros((B, HID), jnp.float32)
    c0 = jnp.zeros((B, HID), jnp.float32)
    h, c = lax.fori_loop(0, tmax, step, (h0, c0))
    y_ref[...] = jnp.dot(h, wy_ref[...],
                         preferred_element_type=jnp.float32) + by_ref[...]
    h_out_ref[...] = h


def _phase_perm(n):
    return jnp.concatenate([jnp.arange(0, n, 2), jnp.arange(1, n, 2)])


def _build_toeplitz1(w, width=40):
    col = jnp.arange(width)[None, :]
    l = jnp.arange(8)[:, None]
    dx = col - 4 * l
    valid = (dx >= 0) & (dx <= 10)
    dxc = jnp.clip(dx, 0, 10)
    g = w[:, :, :, dxc]
    g = jnp.where(valid[None, None, None], g, 0.0)
    g = g.transpose(0, 3, 1, 2, 4)
    g = g[:, _phase_perm(8)].reshape(128, 3 * 11 * width)
    return g.astype(jnp.bfloat16)


def _build_toeplitz2(w):
    col = jnp.arange(16)[None, :]
    j = jnp.arange(14)[:, None]
    dx = col - j
    valid = (dx >= 0) & (dx <= 2)
    dxc = jnp.clip(dx, 0, 2)
    g = w[:, :, :, dxc]
    g = jnp.where(valid[None, None, None], g, 0.0)
    g = g.transpose(0, 3, 1, 2, 4)
    g = g[:, _phase_perm(14)].reshape(448, K2)
    return g.astype(jnp.bfloat16)


@jax.jit
def kernel(datas, img, dataLens, conv1_w, conv1_b, conv2_w, conv2_b,
           lin_w, lin_b, Wf, bf, Wi, bi, Wc, bc, Wo, bo, Wy, by):
    x = img.reshape(BT, 3, IMG, IMG).transpose(1, 2, 3, 0).astype(jnp.bfloat16)
    w1t = _build_toeplitz1(conv1_w, 40)
    w1tb = _build_toeplitz1(conv1_w, 32)
    b1v = jnp.tile(conv1_b[:, None], (1, 8)).reshape(128, 1)
    w2t = _build_toeplitz2(conv2_w)
    b2v = jnp.tile(conv2_b[:, None], (1, 14)).reshape(M2, 1)
    wlt = lin_w.T.astype(jnp.bfloat16)
    blv = lin_b[:, None]

    y1 = pl.pallas_call(
        _conv1_body,
        grid=(NCH, 30),
        in_specs=[
            pl.BlockSpec((3, IMG, IMG, CH), lambda n, s: (0, 0, 0, n)),
            pl.BlockSpec((128, K1G), lambda n, s: (0, 0)),
            pl.BlockSpec((128, 3 * 11 * 32), lambda n, s: (0, 0)),
            pl.BlockSpec((128, 1), lambda n, s: (0, 0)),
        ],
        out_specs=pl.BlockSpec((16, 1, 1, 2, 16, CH),
                               lambda n, s: (0, s % 2, s // 2, 0, 0, n)),
        out_shape=jax.ShapeDtypeStruct((16, 2, 15, 2, 16, BT), jnp.bfloat16),
    )(x, w1t, w1tb, b1v)

    feat_t = pl.pallas_call(
        _cnn_tail_body,
        grid=(NCHT,),
        in_specs=[
            pl.BlockSpec((16, 2, 15, 2, 16, CHT), lambda n: (0, 0, 0, 0, 0, n)),
            pl.BlockSpec((M2, K2), lambda n: (0, 0)),
            pl.BlockSpec((M2, 1), lambda n: (0, 0)),
            pl.BlockSpec((16, 32 * 36), lambda n: (0, 0)),
            pl.BlockSpec((16, 1), lambda n: (0, 0)),
        ],
        out_specs=pl.BlockSpec((16, CHT), lambda n: (0, n)),
        out_shape=jax.ShapeDtypeStruct((16, BT), jnp.float32),
        scratch_shapes=[pltpu.VMEM((16, 16, 16, CHT), jnp.bfloat16),
                        pltpu.VMEM((32, 2, 7, 2, 7, CHT), jnp.bfloat16)],
    )(y1, w2t, b2v, wlt, blv)

    feat = feat_t.T.reshape(B, T, 16)
    xc = jnp.concatenate([feat, datas], axis=2)
    x_tm = xc.transpose(1, 0, 2).reshape(T * B, FEAT)

    wx = jnp.concatenate([Wf[:FEAT], Wi[:FEAT], Wc[:FEAT], Wo[:FEAT]], axis=1)
    wh = jnp.concatenate([Wf[FEAT:], Wi[FEAT:], Wc[FEAT:], Wo[FEAT:]], axis=1)
    bias = jnp.concatenate([bf, bi, bc, bo]).reshape(1, G4)
    lens = dataLens.astype(jnp.int32).reshape(B, 1)

    y, h = pl.pallas_call(
        _lstm_body,
        out_shape=[
            jax.ShapeDtypeStruct((B, OUT), jnp.float32),
            jax.ShapeDtypeStruct((B, HID), jnp.float32),
        ],
        scratch_shapes=[pltpu.VMEM((T * B, G4), jnp.float32)],
    )(lens, x_tm, wx, wh, bias, Wy, by.reshape(1, OUT))
    return (y, h)
